# Initial kernel scaffold; baseline (speedup 1.0000x reference)
import jax
import jax.numpy as jnp
from jax.experimental import pallas as pl


def kernel(x_start, x_noise, timesteps, sqrt_alphas_cumprod, sqrt_one_minus_alphas_cumprod):
    raise NotImplementedError("write your pallas kernel here")


# speedup vs baseline: 3.9230x; 3.9230x over previous
import functools

import jax
import jax.numpy as jnp
from jax import lax
from jax.experimental import pallas as pl
from jax.experimental.pallas import tpu as pltpu
from jax.experimental.pallas import tpu_sc as plsc

_B, _D = 16384, 128
_T = 1000
_TPAD = 1024
_NC, _NS, _L = 2, 16, 16
_NW = _NC * _NS
_BPW = _B // _NW


def _gather_coeffs(table1, table2, timesteps):
    mesh = plsc.VectorSubcoreMesh(core_axis_name="c", subcore_axis_name="s")

    @functools.partial(
        pl.kernel,
        out_type=(
            jax.ShapeDtypeStruct((_B,), jnp.float32),
            jax.ShapeDtypeStruct((_B,), jnp.float32),
        ),
        mesh=mesh,
        compiler_params=pltpu.CompilerParams(needs_layout_passes=False),
        scratch_types=[
            pltpu.VMEM((_TPAD,), jnp.float32),
            pltpu.VMEM((_TPAD,), jnp.float32),
            pltpu.VMEM((_BPW,), jnp.int32),
            pltpu.VMEM((_BPW,), jnp.float32),
            pltpu.VMEM((_BPW,), jnp.float32),
        ],
    )
    def gather_kernel(t1_hbm, t2_hbm, ts_hbm, s1_hbm, s2_hbm,
                      t1_v, t2_v, idx_v, s1_v, s2_v):
        wid = lax.axis_index("s") * _NC + lax.axis_index("c")
        base = wid * _BPW
        pltpu.sync_copy(t1_hbm, t1_v)
        pltpu.sync_copy(t2_hbm, t2_v)
        pltpu.sync_copy(ts_hbm.at[pl.ds(base, _BPW)], idx_v)

        def body(i, carry):
            sl = pl.ds(i * _L, _L)
            idx = idx_v[sl]
            s1_v[sl] = plsc.load_gather(t1_v, [idx])
            s2_v[sl] = plsc.load_gather(t2_v, [idx])
            return carry

        lax.fori_loop(0, _BPW // _L, body, 0, unroll=8)

        pltpu.sync_copy(s1_v, s1_hbm.at[pl.ds(base, _BPW)])
        pltpu.sync_copy(s2_v, s2_hbm.at[pl.ds(base, _BPW)])

    return gather_kernel(table1, table2, timesteps)


def _blend(s1, s2, x_start, x_noise):
    bs = 1024

    def body(s1_ref, s2_ref, xs_ref, xn_ref, o_ref):
        o_ref[...] = s1_ref[...] * xs_ref[...] + s2_ref[...] * xn_ref[...]

    return pl.pallas_call(
        body,
        grid=(_B // bs,),
        in_specs=[
            pl.BlockSpec((bs, 1), lambda i: (i, 0)),
            pl.BlockSpec((bs, 1), lambda i: (i, 0)),
            pl.BlockSpec((bs, _D), lambda i: (i, 0)),
            pl.BlockSpec((bs, _D), lambda i: (i, 0)),
        ],
        out_specs=pl.BlockSpec((bs, _D), lambda i: (i, 0)),
        out_shape=jax.ShapeDtypeStruct((_B, _D), jnp.float32),
    )(s1.reshape(_B, 1), s2.reshape(_B, 1), x_start, x_noise)


def kernel(x_start, x_noise, timesteps, sqrt_alphas_cumprod,
           sqrt_one_minus_alphas_cumprod):
    t1 = jnp.pad(sqrt_alphas_cumprod, (0, _TPAD - _T))
    t2 = jnp.pad(sqrt_one_minus_alphas_cumprod, (0, _TPAD - _T))
    s1, s2 = _gather_coeffs(t1, t2, timesteps)
    return _blend(s1, s2, x_start, x_noise)


# speedup vs baseline: 5.1793x; 1.3202x over previous
import functools

import jax
import jax.numpy as jnp
from jax import lax
from jax.experimental import pallas as pl
from jax.experimental.pallas import tpu as pltpu
from jax.experimental.pallas import tpu_sc as plsc

_B, _D = 16384, 128
_T = 1000
_TPAD = 1024
_NC, _NS, _L = 2, 16, 16
_NW = _NC * _NS
_BPW = _B // _NW


def _gather_coeffs(table1, table2, timesteps):
    mesh = plsc.VectorSubcoreMesh(core_axis_name="c", subcore_axis_name="s")

    @functools.partial(
        pl.kernel,
        out_type=(
            jax.ShapeDtypeStruct((_B,), jnp.float32),
            jax.ShapeDtypeStruct((_B,), jnp.float32),
        ),
        mesh=mesh,
        compiler_params=pltpu.CompilerParams(needs_layout_passes=False),
        scratch_types=[
            pltpu.VMEM((_TPAD,), jnp.float32),
            pltpu.VMEM((_TPAD,), jnp.float32),
            pltpu.VMEM((_BPW,), jnp.int32),
            pltpu.VMEM((_BPW,), jnp.float32),
            pltpu.VMEM((_BPW,), jnp.float32),
        ],
    )
    def gather_kernel(t1_hbm, t2_hbm, ts_hbm, s1_hbm, s2_hbm,
                      t1_v, t2_v, idx_v, s1_v, s2_v):
        wid = lax.axis_index("s") * _NC + lax.axis_index("c")
        base = wid * _BPW
        pltpu.sync_copy(t1_hbm, t1_v)
        pltpu.sync_copy(t2_hbm, t2_v)
        pltpu.sync_copy(ts_hbm.at[pl.ds(base, _BPW)], idx_v)

        def body(i, carry):
            sl = pl.ds(i * _L, _L)
            idx = idx_v[sl]
            s1_v[sl] = plsc.load_gather(t1_v, [idx])
            s2_v[sl] = plsc.load_gather(t2_v, [idx])
            return carry

        lax.fori_loop(0, _BPW // _L, body, 0, unroll=8)

        pltpu.sync_copy(s1_v, s1_hbm.at[pl.ds(base, _BPW)])
        pltpu.sync_copy(s2_v, s2_hbm.at[pl.ds(base, _BPW)])

    return gather_kernel(table1, table2, timesteps)


def _blend(s1, s2, x_start, x_noise):
    bs = 1024

    def body(s1_ref, s2_ref, xs_ref, xn_ref, o_ref):
        c1 = s1_ref[...].reshape(bs, 1)
        c2 = s2_ref[...].reshape(bs, 1)
        o_ref[...] = c1 * xs_ref[...] + c2 * xn_ref[...]

    return pl.pallas_call(
        body,
        grid=(_B // bs,),
        in_specs=[
            pl.BlockSpec((bs,), lambda i: (i,)),
            pl.BlockSpec((bs,), lambda i: (i,)),
            pl.BlockSpec((bs, _D), lambda i: (i, 0)),
            pl.BlockSpec((bs, _D), lambda i: (i, 0)),
        ],
        out_specs=pl.BlockSpec((bs, _D), lambda i: (i, 0)),
        out_shape=jax.ShapeDtypeStruct((_B, _D), jnp.float32),
    )(s1, s2, x_start, x_noise)


def kernel(x_start, x_noise, timesteps, sqrt_alphas_cumprod,
           sqrt_one_minus_alphas_cumprod):
    t1 = jnp.pad(sqrt_alphas_cumprod, (0, _TPAD - _T))
    t2 = jnp.pad(sqrt_one_minus_alphas_cumprod, (0, _TPAD - _T))
    s1, s2 = _gather_coeffs(t1, t2, timesteps)
    return _blend(s1, s2, x_start, x_noise)
